# baseline (speedup 1.0000x reference)
import jax
import jax.numpy as jnp
import numpy as np
from functools import partial
from jax import lax
from jax.experimental import pallas as pl
from jax.experimental.pallas import tpu as pltpu
from jax.experimental.pallas import tpu_sc as plsc

T = 3
N = 4096
E = 131072
X_DIM = 128
H_DIM = 128
Z_DIM = 64
ALPHA = 0.2

_R = 512
_C = 1024
_ROWS = 256


def _gat_body(acts, m_ref, ht_ref, si_ref, sj_ref, out_ref, rs_ref):
    cb = pl.program_id(1)
    ncb = pl.num_programs(1)
    G = ht_ref.shape[0]

    @pl.when(cb == 0)
    def _():
        out_ref[...] = jnp.zeros_like(out_ref)
        rs_ref[...] = jnp.zeros_like(rs_ref)

    m = m_ref[...]
    for k in range(G):
        s = si_ref[k] + sj_ref[k]
        w = jnp.exp(jnp.where(s > 0, -s, -ALPHA * s))
        p = m * w
        out_ref[k] += jnp.dot(p, ht_ref[k], preferred_element_type=jnp.float32)
        rs_ref[k] += jnp.sum(p, axis=1, keepdims=True)

    @pl.when(cb == ncb - 1)
    def _():
        for k in range(G):
            out_ref[k] = acts[k](out_ref[k] / rs_ref[k])


def _gat_pass(M, HT, SI, SJ, acts):
    G, n, D = HT.shape
    grid = (N // _R, N // _C)
    return pl.pallas_call(
        partial(_gat_body, acts),
        grid=grid,
        in_specs=[
            pl.BlockSpec((_R, _C), lambda rb, cb: (rb, cb)),
            pl.BlockSpec((G, _C, D), lambda rb, cb: (0, cb, 0)),
            pl.BlockSpec((G, _R, 1), lambda rb, cb: (0, rb, 0)),
            pl.BlockSpec((G, 1, _C), lambda rb, cb: (0, 0, cb)),
        ],
        out_specs=pl.BlockSpec((G, _R, D), lambda rb, cb: (0, rb, 0)),
        out_shape=jax.ShapeDtypeStruct((G, N, D), jnp.float32),
        scratch_shapes=[pltpu.VMEM((G, _R, 1), jnp.float32)],
    )(M, HT, SI, SJ)


def _scores(HT, A):
    G, n, D = HT.shape
    si = jnp.einsum('gnd,gd->gn', HT, A[:, :D])
    sj = jnp.einsum('gnd,gd->gn', HT, A[:, D:])
    return si[:, :, None], sj[:, None, :]


def _nll_body(adj_ref, h1_ref, h2_ref, out_ref):
    t = pl.program_id(0)
    b = pl.program_id(1)

    @pl.when(jnp.logical_and(t == 0, b == 0))
    def _():
        out_ref[...] = jnp.zeros_like(out_ref)

    s = h1_ref[0] + h2_ref[0]
    p = jax.nn.sigmoid(s)
    p = jnp.clip(p, 1e-7, 1.0 - 1e-7)
    adj = adj_ref[0]
    term = adj * jnp.log(p) + (1.0 - adj) * jnp.log(1.0 - p)
    scale = 1.0 / (float(N) * float(N) * 8.0 * 128.0)
    out_ref[...] += jnp.sum(term) * scale


def _nll_all(adj, h1, h2):
    nb = N // _ROWS
    out = pl.pallas_call(
        _nll_body,
        grid=(T, nb),
        in_specs=[
            pl.BlockSpec((1, _ROWS, N), lambda t, b: (t, b, 0)),
            pl.BlockSpec((1, _ROWS, 1), lambda t, b: (t, b, 0)),
            pl.BlockSpec((1, 1, N), lambda t, b: (t, 0, 0)),
        ],
        out_specs=pl.BlockSpec((8, 128), lambda t, b: (0, 0)),
        out_shape=jax.ShapeDtypeStruct((8, 128), jnp.float32),
    )(adj, h1, h2)
    return -jnp.sum(out)


def _identity(v):
    return v


def _elu(v):
    return jnp.where(v > 0, v, jnp.exp(jnp.minimum(v, 0.0)) - 1.0)


def _softplus(v):
    return jnp.maximum(v, 0.0) + jnp.log(1.0 + jnp.exp(-jnp.abs(v)))


def _kld_gauss(m1, s1, m2, s2):
    eps = 1e-8
    kld = 2.0 * jnp.log(s2 + eps) - 2.0 * jnp.log(s1 + eps) + (s1 ** 2 + (m1 - m2) ** 2) / ((s2 + eps) ** 2) - 1.0
    return (0.5 / m1.shape[0]) * jnp.sum(kld)


_EPT = E // 16
_RWIN = 256
_NPASS = N // (2 * _RWIN)
_ACC = _RWIN * N
_SHARE = _ACC // 16


def _mbuild_body(e0_hbm, e1_hbm, m_hbm, e0_v, e1_v, idx_v, val_v, zero_v, acc_sh):
    c = lax.axis_index("c")
    s = lax.axis_index("s")
    pltpu.sync_copy(e0_hbm.at[pl.ds(s * _EPT, _EPT)], e0_v)
    pltpu.sync_copy(e1_hbm.at[pl.ds(s * _EPT, _EPT)], e1_v)

    def zfill(i, _):
        zero_v[pl.ds(i * 16, 16)] = jnp.zeros((16,), jnp.float32)
        return 0
    lax.fori_loop(0, zero_v.shape[0] // 16, zfill, 0)

    for p in range(_NPASS):
        base = (p * 2 + c) * _RWIN

        def zcopy(i, _):
            pltpu.sync_copy(zero_v, acc_sh.at[pl.ds(s * _SHARE + i * 8192, 8192)])
            return 0
        lax.fori_loop(0, _SHARE // 8192, zcopy, 0)
        plsc.subcore_barrier()

        def estep(i, _):
            e0v = e0_v[pl.ds(i * 16, 16)]
            e1v = e1_v[pl.ds(i * 16, 16)]
            rel = e0v - base
            mask = (rel >= 0) & (rel < _RWIN)
            flat = rel * N + e1v
            idx_v[pl.ds(i * 16, 16)] = jnp.where(mask, flat, 0)
            val_v[pl.ds(i * 16, 16)] = jnp.where(mask, 1.0, 0.0)
            return 0
        lax.fori_loop(0, _EPT // 16, estep, 0)

        rel_d = s * 16 + lax.iota(jnp.int32, 16)
        idx_v[pl.ds(_EPT, 16)] = rel_d * N + (base + rel_d)
        val_v[pl.ds(_EPT, 16)] = jnp.ones((16,), jnp.float32)

        pltpu.sync_copy(val_v, acc_sh.at[idx_v], add=True)
        plsc.subcore_barrier()
        pltpu.sync_copy(acc_sh.at[pl.ds(s * _SHARE, _SHARE)],
                        m_hbm.at[pl.ds(base * N + s * _SHARE, _SHARE)])
        plsc.subcore_barrier()


_mbuild = partial(
    pl.kernel,
    out_type=jax.ShapeDtypeStruct((N * N,), jnp.float32),
    mesh=plsc.VectorSubcoreMesh(core_axis_name="c", subcore_axis_name="s"),
    scratch_types=[
        pltpu.VMEM((_EPT,), jnp.int32),
        pltpu.VMEM((_EPT,), jnp.int32),
        pltpu.VMEM((_EPT + 16,), jnp.int32),
        pltpu.VMEM((_EPT + 16,), jnp.float32),
        pltpu.VMEM((8192,), jnp.float32),
        pltpu.VMEM_SHARED((_ACC,), jnp.float32),
    ],
)(_mbuild_body)


def _build_m(edge):
    return _mbuild(edge[0], edge[1]).reshape(N, N)


def kernel(x, edge_idx_list, adj_orig_dense_list, phi_x_W, phi_x_b, phi_z_W, phi_z_b, enc_W, enc_a, enc_mu_W, enc_mu_a, enc_lv_W, enc_lv_a, prior_W, prior_b, prior_mu_W, prior_mu_b, prior_lv_W, prior_lv_b, lstm_x_W, lstm_x_a, lstm_h_W, lstm_h_a, dec_a):
    h = jnp.zeros((N, H_DIM), dtype=jnp.float32)
    c = jnp.zeros((N, H_DIM), dtype=jnp.float32)
    kld = jnp.float32(0.0)
    h1s, h2s = [], []
    for t in range(T):
        M = _build_m(edge_idx_list[t])
        phi_x_t = jax.nn.relu(x[t] @ phi_x_W + phi_x_b)

        h_enc = jnp.concatenate([phi_x_t, h], axis=1) @ enc_W
        HT1 = h_enc[None]
        SI1, SJ1 = _scores(HT1, enc_a)
        enc_t = _gat_pass(M, HT1, SI1, SJ1, [_elu])[0]

        HT2 = jnp.stack([enc_t @ enc_mu_W, enc_t @ enc_lv_W])
        A2 = jnp.stack([enc_mu_a[0], enc_lv_a[0]])
        SI2, SJ2 = _scores(HT2, A2)
        o2 = _gat_pass(M, HT2, SI2, SJ2, [_identity, _softplus])
        enc_mu_t, enc_std_t = o2[0], o2[1]

        prior_t = jax.nn.elu(h @ prior_W + prior_b)
        prior_mu_t = prior_t @ prior_mu_W + prior_mu_b
        prior_std_t = jax.nn.softplus(prior_t @ prior_lv_W + prior_lv_b)
        eps = jax.random.normal(jax.random.fold_in(jax.random.key(7), t), enc_mu_t.shape, dtype=jnp.float32)
        z_t = enc_mu_t + eps * enc_std_t
        phi_z_t = jax.nn.relu(z_t @ phi_z_W + phi_z_b)
        x_lstm = jnp.concatenate([phi_x_t, phi_z_t], axis=1)

        HT3 = jnp.concatenate([
            jnp.einsum('nk,gkd->gnd', x_lstm, lstm_x_W),
            jnp.einsum('nk,gkd->gnd', h, lstm_h_W),
        ])
        A3 = jnp.concatenate([lstm_x_a[:, 0, :], lstm_h_a[:, 0, :]])
        SI3, SJ3 = _scores(HT3, A3)
        o3 = _gat_pass(M, HT3, SI3, SJ3, [_identity] * 8)

        ig = jax.nn.sigmoid(o3[0] + o3[4])
        fg = jax.nn.sigmoid(o3[1] + o3[5])
        og = jax.nn.sigmoid(o3[2] + o3[6])
        ct = jnp.tanh(o3[3] + o3[7])
        c = fg * c + ig * ct
        h = og * jnp.tanh(c)
        kld = kld + _kld_gauss(enc_mu_t, enc_std_t, prior_mu_t, prior_std_t)
        h1s.append(z_t @ dec_a[:Z_DIM, :])
        h2s.append((z_t @ dec_a[Z_DIM:, :]).T)

    h1 = jnp.stack(h1s)
    h2 = jnp.stack(h2s)
    nll = _nll_all(adj_orig_dense_list, h1, h2)
    return jnp.stack([kld, nll])


# speedup vs baseline: 2.6221x; 2.6221x over previous
import jax
import jax.numpy as jnp
import numpy as np
from functools import partial
from jax import lax
from jax.experimental import pallas as pl
from jax.experimental.pallas import tpu as pltpu
from jax.experimental.pallas import tpu_sc as plsc

T = 3
N = 4096
E = 131072
X_DIM = 128
H_DIM = 128
Z_DIM = 64
ALPHA = 0.2

_R = 512
_C = 1024
_ROWS = 256


def _gat_body(acts, m_ref, ht_ref, si_ref, sj_ref, out_ref, rs_ref):
    cb = pl.program_id(1)
    ncb = pl.num_programs(1)
    G = ht_ref.shape[0]

    @pl.when(cb == 0)
    def _():
        out_ref[...] = jnp.zeros_like(out_ref)
        rs_ref[...] = jnp.zeros_like(rs_ref)

    m = m_ref[...]
    for k in range(G):
        s = si_ref[k] + sj_ref[k]
        w = jnp.exp(jnp.where(s > 0, -s, -ALPHA * s))
        p = m * w
        out_ref[k] += jnp.dot(p, ht_ref[k], preferred_element_type=jnp.float32)
        rs_ref[k] += jnp.sum(p, axis=1, keepdims=True)

    @pl.when(cb == ncb - 1)
    def _():
        for k in range(G):
            out_ref[k] = acts[k](out_ref[k] / rs_ref[k])


def _gat_pass(M, HT, SI, SJ, acts):
    G, n, D = HT.shape
    grid = (N // _R, N // _C)
    return pl.pallas_call(
        partial(_gat_body, acts),
        grid=grid,
        in_specs=[
            pl.BlockSpec((_R, _C), lambda rb, cb: (rb, cb)),
            pl.BlockSpec((G, _C, D), lambda rb, cb: (0, cb, 0)),
            pl.BlockSpec((G, _R, 1), lambda rb, cb: (0, rb, 0)),
            pl.BlockSpec((G, 1, _C), lambda rb, cb: (0, 0, cb)),
        ],
        out_specs=pl.BlockSpec((G, _R, D), lambda rb, cb: (0, rb, 0)),
        out_shape=jax.ShapeDtypeStruct((G, N, D), jnp.float32),
        scratch_shapes=[pltpu.VMEM((G, _R, 1), jnp.float32)],
    )(M, HT, SI, SJ)


def _scores(HT, A):
    G, n, D = HT.shape
    si = jnp.einsum('gnd,gd->gn', HT, A[:, :D])
    sj = jnp.einsum('gnd,gd->gn', HT, A[:, D:])
    return si[:, :, None], sj[:, None, :]


def _nll_body(adj_ref, h1_ref, h2_ref, out_ref):
    t = pl.program_id(0)
    b = pl.program_id(1)

    @pl.when(jnp.logical_and(t == 0, b == 0))
    def _():
        out_ref[...] = jnp.zeros_like(out_ref)

    s = h1_ref[0] + h2_ref[0]
    p = jax.nn.sigmoid(s)
    p = jnp.clip(p, 1e-7, 1.0 - 1e-7)
    adj = adj_ref[0]
    term = adj * jnp.log(p) + (1.0 - adj) * jnp.log(1.0 - p)
    scale = 1.0 / (float(N) * float(N) * 8.0 * 128.0)
    out_ref[...] += jnp.sum(term) * scale


def _nll_all(adj, h1, h2):
    nb = N // _ROWS
    out = pl.pallas_call(
        _nll_body,
        grid=(T, nb),
        in_specs=[
            pl.BlockSpec((1, _ROWS, N), lambda t, b: (t, b, 0)),
            pl.BlockSpec((1, _ROWS, 1), lambda t, b: (t, b, 0)),
            pl.BlockSpec((1, 1, N), lambda t, b: (t, 0, 0)),
        ],
        out_specs=pl.BlockSpec((8, 128), lambda t, b: (0, 0)),
        out_shape=jax.ShapeDtypeStruct((8, 128), jnp.float32),
    )(adj, h1, h2)
    return -jnp.sum(out)


def _identity(v):
    return v


def _elu(v):
    return jnp.where(v > 0, v, jnp.exp(jnp.minimum(v, 0.0)) - 1.0)


def _softplus(v):
    return jnp.maximum(v, 0.0) + jnp.log(1.0 + jnp.exp(-jnp.abs(v)))


def _kld_gauss(m1, s1, m2, s2):
    eps = 1e-8
    kld = 2.0 * jnp.log(s2 + eps) - 2.0 * jnp.log(s1 + eps) + (s1 ** 2 + (m1 - m2) ** 2) / ((s2 + eps) ** 2) - 1.0
    return (0.5 / m1.shape[0]) * jnp.sum(kld)


_EPT = E // 16
_RWIN = 256
_NPASS = N // (2 * _RWIN)
_ACC = _RWIN * N
_SHARE = _ACC // 16


def _mbuild_body(e0_hbm, e1_hbm, m_hbm, e0_v, e1_v, idx_v, val_v, zero_v, acc_sh):
    c = lax.axis_index("c")
    s = lax.axis_index("s")
    pltpu.sync_copy(e0_hbm.at[pl.ds(s * _EPT, _EPT)], e0_v)
    pltpu.sync_copy(e1_hbm.at[pl.ds(s * _EPT, _EPT)], e1_v)

    def zfill(i, _):
        zero_v[pl.ds(i * 16, 16)] = jnp.zeros((16,), jnp.float32)
        return 0
    lax.fori_loop(0, zero_v.shape[0] // 16, zfill, 0)

    for p in range(_NPASS):
        base = (p * 2 + c) * _RWIN

        def zcopy(i, _):
            pltpu.sync_copy(zero_v, acc_sh.at[pl.ds(s * _SHARE + i * 8192, 8192)])
            return 0
        lax.fori_loop(0, _SHARE // 8192, zcopy, 0)
        plsc.subcore_barrier()

        lanes = lax.iota(jnp.int32, 16)

        def estep(i, _):
            e0v = e0_v[pl.ds(i * 16, 16)]
            e1v = e1_v[pl.ds(i * 16, 16)]
            rel = e0v - base
            mask = (rel >= 0) & (rel < _RWIN)
            flat = rel * N + e1v
            junk = s * _EPT + i * 16 + lanes
            idx_v[pl.ds(i * 16, 16)] = jnp.where(mask, flat, junk)
            val_v[pl.ds(i * 16, 16)] = jnp.where(mask, 1.0, 0.0)
            return 0
        lax.fori_loop(0, _EPT // 16, estep, 0)

        rel_d = s * 16 + lax.iota(jnp.int32, 16)
        idx_v[pl.ds(_EPT, 16)] = rel_d * N + (base + rel_d)
        val_v[pl.ds(_EPT, 16)] = jnp.ones((16,), jnp.float32)

        pltpu.sync_copy(val_v, acc_sh.at[idx_v], add=True)
        plsc.subcore_barrier()
        pltpu.sync_copy(acc_sh.at[pl.ds(s * _SHARE, _SHARE)],
                        m_hbm.at[pl.ds(base * N + s * _SHARE, _SHARE)])
        plsc.subcore_barrier()


_mbuild = partial(
    pl.kernel,
    out_type=jax.ShapeDtypeStruct((N * N,), jnp.float32),
    mesh=plsc.VectorSubcoreMesh(core_axis_name="c", subcore_axis_name="s"),
    scratch_types=[
        pltpu.VMEM((_EPT,), jnp.int32),
        pltpu.VMEM((_EPT,), jnp.int32),
        pltpu.VMEM((_EPT + 16,), jnp.int32),
        pltpu.VMEM((_EPT + 16,), jnp.float32),
        pltpu.VMEM((8192,), jnp.float32),
        pltpu.VMEM_SHARED((_ACC,), jnp.float32),
    ],
)(_mbuild_body)


def _build_m(edge):
    return _mbuild(edge[0], edge[1]).reshape(N, N)


def kernel(x, edge_idx_list, adj_orig_dense_list, phi_x_W, phi_x_b, phi_z_W, phi_z_b, enc_W, enc_a, enc_mu_W, enc_mu_a, enc_lv_W, enc_lv_a, prior_W, prior_b, prior_mu_W, prior_mu_b, prior_lv_W, prior_lv_b, lstm_x_W, lstm_x_a, lstm_h_W, lstm_h_a, dec_a):
    h = jnp.zeros((N, H_DIM), dtype=jnp.float32)
    c = jnp.zeros((N, H_DIM), dtype=jnp.float32)
    kld = jnp.float32(0.0)
    h1s, h2s = [], []
    for t in range(T):
        M = _build_m(edge_idx_list[t])
        phi_x_t = jax.nn.relu(x[t] @ phi_x_W + phi_x_b)

        h_enc = jnp.concatenate([phi_x_t, h], axis=1) @ enc_W
        HT1 = h_enc[None]
        SI1, SJ1 = _scores(HT1, enc_a)
        enc_t = _gat_pass(M, HT1, SI1, SJ1, [_elu])[0]

        HT2 = jnp.stack([enc_t @ enc_mu_W, enc_t @ enc_lv_W])
        A2 = jnp.stack([enc_mu_a[0], enc_lv_a[0]])
        SI2, SJ2 = _scores(HT2, A2)
        o2 = _gat_pass(M, HT2, SI2, SJ2, [_identity, _softplus])
        enc_mu_t, enc_std_t = o2[0], o2[1]

        prior_t = jax.nn.elu(h @ prior_W + prior_b)
        prior_mu_t = prior_t @ prior_mu_W + prior_mu_b
        prior_std_t = jax.nn.softplus(prior_t @ prior_lv_W + prior_lv_b)
        eps = jax.random.normal(jax.random.fold_in(jax.random.key(7), t), enc_mu_t.shape, dtype=jnp.float32)
        z_t = enc_mu_t + eps * enc_std_t
        phi_z_t = jax.nn.relu(z_t @ phi_z_W + phi_z_b)
        x_lstm = jnp.concatenate([phi_x_t, phi_z_t], axis=1)

        HT3 = jnp.concatenate([
            jnp.einsum('nk,gkd->gnd', x_lstm, lstm_x_W),
            jnp.einsum('nk,gkd->gnd', h, lstm_h_W),
        ])
        A3 = jnp.concatenate([lstm_x_a[:, 0, :], lstm_h_a[:, 0, :]])
        SI3, SJ3 = _scores(HT3, A3)
        o3 = _gat_pass(M, HT3, SI3, SJ3, [_identity] * 8)

        ig = jax.nn.sigmoid(o3[0] + o3[4])
        fg = jax.nn.sigmoid(o3[1] + o3[5])
        og = jax.nn.sigmoid(o3[2] + o3[6])
        ct = jnp.tanh(o3[3] + o3[7])
        c = fg * c + ig * ct
        h = og * jnp.tanh(c)
        kld = kld + _kld_gauss(enc_mu_t, enc_std_t, prior_mu_t, prior_std_t)
        h1s.append(z_t @ dec_a[:Z_DIM, :])
        h2s.append((z_t @ dec_a[Z_DIM:, :]).T)

    h1 = jnp.stack(h1s)
    h2 = jnp.stack(h2s)
    nll = _nll_all(adj_orig_dense_list, h1, h2)
    return jnp.stack([kld, nll])


# speedup vs baseline: 2.8973x; 1.1049x over previous
import jax
import jax.numpy as jnp
import numpy as np
from functools import partial
from jax import lax
from jax.experimental import pallas as pl
from jax.experimental.pallas import tpu as pltpu
from jax.experimental.pallas import tpu_sc as plsc

T = 3
N = 4096
E = 131072
X_DIM = 128
H_DIM = 128
Z_DIM = 64
ALPHA = 0.2

_R = 512
_C = 1024
_ROWS = 256


def _gat_body(acts, m_ref, ht_ref, a1_ref, a2_ref, b1_ref, b2_ref, out_ref, rs_ref):
    cb = pl.program_id(1)
    ncb = pl.num_programs(1)
    G = ht_ref.shape[0]

    @pl.when(cb == 0)
    def _():
        out_ref[...] = jnp.zeros_like(out_ref)
        rs_ref[...] = jnp.zeros_like(rs_ref)

    m = m_ref[...]
    for k in range(G):
        w = jnp.minimum(a1_ref[k] * b1_ref[k], a2_ref[k] * b2_ref[k])
        p = m * w
        out_ref[k] += jnp.dot(p.astype(jnp.bfloat16), ht_ref[k],
                              preferred_element_type=jnp.float32)
        rs_ref[k] += jnp.sum(p, axis=1, keepdims=True)

    @pl.when(cb == ncb - 1)
    def _():
        for k in range(G):
            out_ref[k] = acts[k](out_ref[k] / rs_ref[k])


def _gat_pass(M, HT, FA, acts):
    G, n, D = HT.shape
    A1, A2, B1, B2 = FA
    grid = (N // _R, N // _C)
    row_spec = pl.BlockSpec((G, _R, 1), lambda rb, cb: (0, rb, 0))
    col_spec = pl.BlockSpec((G, 1, _C), lambda rb, cb: (0, 0, cb))
    return pl.pallas_call(
        partial(_gat_body, acts),
        grid=grid,
        in_specs=[
            pl.BlockSpec((_R, _C), lambda rb, cb: (rb, cb)),
            pl.BlockSpec((G, _C, D), lambda rb, cb: (0, cb, 0)),
            row_spec, row_spec, col_spec, col_spec,
        ],
        out_specs=pl.BlockSpec((G, _R, D), lambda rb, cb: (0, rb, 0)),
        out_shape=jax.ShapeDtypeStruct((G, N, D), jnp.float32),
        scratch_shapes=[pltpu.VMEM((G, _R, 1), jnp.float32)],
    )(M, HT, A1, A2, B1, B2)


def _factors(HT, A):
    G, n, D = HT.shape
    si = jnp.clip(jnp.einsum('gnd,gd->gn', HT, A[:, :D]), -60.0, 60.0)
    sj = jnp.clip(jnp.einsum('gnd,gd->gn', HT, A[:, D:]), -60.0, 60.0)
    a1 = jnp.exp(-si)[:, :, None]
    a2 = jnp.exp(-ALPHA * si)[:, :, None]
    b1 = jnp.exp(-sj)[:, None, :]
    b2 = jnp.exp(-ALPHA * sj)[:, None, :]
    return a1, a2, b1, b2


def _nll_body(adj_ref, h1_ref, h2_ref, out_ref):
    t = pl.program_id(0)
    b = pl.program_id(1)

    @pl.when(jnp.logical_and(t == 0, b == 0))
    def _():
        out_ref[...] = jnp.zeros_like(out_ref)

    s = h1_ref[0] + h2_ref[0]
    p = jax.nn.sigmoid(s)
    p = jnp.clip(p, 1e-7, 1.0 - 1e-7)
    adj = adj_ref[0]
    term = adj * jnp.log(p) + (1.0 - adj) * jnp.log(1.0 - p)
    scale = 1.0 / (float(N) * float(N) * 8.0 * 128.0)
    out_ref[...] += jnp.sum(term) * scale


def _nll_all(adj, h1, h2):
    nb = N // _ROWS
    out = pl.pallas_call(
        _nll_body,
        grid=(T, nb),
        in_specs=[
            pl.BlockSpec((1, _ROWS, N), lambda t, b: (t, b, 0)),
            pl.BlockSpec((1, _ROWS, 1), lambda t, b: (t, b, 0)),
            pl.BlockSpec((1, 1, N), lambda t, b: (t, 0, 0)),
        ],
        out_specs=pl.BlockSpec((8, 128), lambda t, b: (0, 0)),
        out_shape=jax.ShapeDtypeStruct((8, 128), jnp.float32),
    )(adj, h1, h2)
    return -jnp.sum(out)


def _identity(v):
    return v


def _elu(v):
    return jnp.where(v > 0, v, jnp.exp(jnp.minimum(v, 0.0)) - 1.0)


def _softplus(v):
    return jnp.maximum(v, 0.0) + jnp.log(1.0 + jnp.exp(-jnp.abs(v)))


def _kld_gauss(m1, s1, m2, s2):
    eps = 1e-8
    kld = 2.0 * jnp.log(s2 + eps) - 2.0 * jnp.log(s1 + eps) + (s1 ** 2 + (m1 - m2) ** 2) / ((s2 + eps) ** 2) - 1.0
    return (0.5 / m1.shape[0]) * jnp.sum(kld)


_EPT = E // 16
_RWIN = 256
_NPASS = N // (2 * _RWIN)
_ACC = _RWIN * N
_SHARE = _ACC // 16


def _mbuild_body(e0_hbm, e1_hbm, m_hbm, e0_v, e1_v, idx_v, val_v, zero_v, acc_sh):
    c = lax.axis_index("c")
    s = lax.axis_index("s")
    pltpu.sync_copy(e0_hbm.at[pl.ds(s * _EPT, _EPT)], e0_v)
    pltpu.sync_copy(e1_hbm.at[pl.ds(s * _EPT, _EPT)], e1_v)

    def zfill(i, _):
        zero_v[pl.ds(i * 16, 16)] = jnp.zeros((16,), jnp.float32)
        return 0
    lax.fori_loop(0, zero_v.shape[0] // 16, zfill, 0)

    for p in range(_NPASS):
        base = (p * 2 + c) * _RWIN

        def zcopy(i, _):
            pltpu.sync_copy(zero_v, acc_sh.at[pl.ds(s * _SHARE + i * 8192, 8192)])
            return 0
        lax.fori_loop(0, _SHARE // 8192, zcopy, 0)
        plsc.subcore_barrier()

        lanes = lax.iota(jnp.int32, 16)

        def estep(i, _):
            e0v = e0_v[pl.ds(i * 16, 16)]
            e1v = e1_v[pl.ds(i * 16, 16)]
            rel = e0v - base
            mask = (rel >= 0) & (rel < _RWIN)
            flat = rel * N + e1v
            junk = s * _EPT + i * 16 + lanes
            idx_v[pl.ds(i * 16, 16)] = jnp.where(mask, flat, junk)
            val_v[pl.ds(i * 16, 16)] = jnp.where(mask, 1.0, 0.0)
            return 0
        lax.fori_loop(0, _EPT // 16, estep, 0)

        rel_d = s * 16 + lax.iota(jnp.int32, 16)
        idx_v[pl.ds(_EPT, 16)] = rel_d * N + (base + rel_d)
        val_v[pl.ds(_EPT, 16)] = jnp.ones((16,), jnp.float32)

        pltpu.sync_copy(val_v, acc_sh.at[idx_v], add=True)
        plsc.subcore_barrier()
        pltpu.sync_copy(acc_sh.at[pl.ds(s * _SHARE, _SHARE)],
                        m_hbm.at[pl.ds(base * N + s * _SHARE, _SHARE)])
        plsc.subcore_barrier()


_mbuild = partial(
    pl.kernel,
    out_type=jax.ShapeDtypeStruct((N * N,), jnp.float32),
    mesh=plsc.VectorSubcoreMesh(core_axis_name="c", subcore_axis_name="s"),
    scratch_types=[
        pltpu.VMEM((_EPT,), jnp.int32),
        pltpu.VMEM((_EPT,), jnp.int32),
        pltpu.VMEM((_EPT + 16,), jnp.int32),
        pltpu.VMEM((_EPT + 16,), jnp.float32),
        pltpu.VMEM((8192,), jnp.float32),
        pltpu.VMEM_SHARED((_ACC,), jnp.float32),
    ],
)(_mbuild_body)


def _build_m(edge):
    return _mbuild(edge[0], edge[1]).reshape(N, N)


def kernel(x, edge_idx_list, adj_orig_dense_list, phi_x_W, phi_x_b, phi_z_W, phi_z_b, enc_W, enc_a, enc_mu_W, enc_mu_a, enc_lv_W, enc_lv_a, prior_W, prior_b, prior_mu_W, prior_mu_b, prior_lv_W, prior_lv_b, lstm_x_W, lstm_x_a, lstm_h_W, lstm_h_a, dec_a):
    h = jnp.zeros((N, H_DIM), dtype=jnp.float32)
    c = jnp.zeros((N, H_DIM), dtype=jnp.float32)
    kld = jnp.float32(0.0)
    h1s, h2s = [], []
    for t in range(T):
        M = _build_m(edge_idx_list[t])
        phi_x_t = jax.nn.relu(x[t] @ phi_x_W + phi_x_b)

        h_enc = jnp.concatenate([phi_x_t, h], axis=1) @ enc_W
        HT1 = h_enc[None]
        enc_t = _gat_pass(M, HT1.astype(jnp.bfloat16), _factors(HT1, enc_a), [_elu])[0]

        HT2 = jnp.stack([enc_t @ enc_mu_W, enc_t @ enc_lv_W])
        A2 = jnp.stack([enc_mu_a[0], enc_lv_a[0]])
        o2 = _gat_pass(M, HT2.astype(jnp.bfloat16), _factors(HT2, A2), [_identity, _softplus])
        enc_mu_t, enc_std_t = o2[0], o2[1]

        prior_t = jax.nn.elu(h @ prior_W + prior_b)
        prior_mu_t = prior_t @ prior_mu_W + prior_mu_b
        prior_std_t = jax.nn.softplus(prior_t @ prior_lv_W + prior_lv_b)
        eps = jax.random.normal(jax.random.fold_in(jax.random.key(7), t), enc_mu_t.shape, dtype=jnp.float32)
        z_t = enc_mu_t + eps * enc_std_t
        phi_z_t = jax.nn.relu(z_t @ phi_z_W + phi_z_b)
        x_lstm = jnp.concatenate([phi_x_t, phi_z_t], axis=1)

        HT3 = jnp.concatenate([
            jnp.einsum('nk,gkd->gnd', x_lstm, lstm_x_W),
            jnp.einsum('nk,gkd->gnd', h, lstm_h_W),
        ])
        A3 = jnp.concatenate([lstm_x_a[:, 0, :], lstm_h_a[:, 0, :]])
        o3 = _gat_pass(M, HT3.astype(jnp.bfloat16), _factors(HT3, A3), [_identity] * 8)

        ig = jax.nn.sigmoid(o3[0] + o3[4])
        fg = jax.nn.sigmoid(o3[1] + o3[5])
        og = jax.nn.sigmoid(o3[2] + o3[6])
        ct = jnp.tanh(o3[3] + o3[7])
        c = fg * c + ig * ct
        h = og * jnp.tanh(c)
        kld = kld + _kld_gauss(enc_mu_t, enc_std_t, prior_mu_t, prior_std_t)
        h1s.append(z_t @ dec_a[:Z_DIM, :])
        h2s.append((z_t @ dec_a[Z_DIM:, :]).T)

    h1 = jnp.stack(h1s)
    h2 = jnp.stack(h2s)
    nll = _nll_all(adj_orig_dense_list, h1, h2)
    return jnp.stack([kld, nll])


# speedup vs baseline: 2.9481x; 1.0175x over previous
import jax
import jax.numpy as jnp
import numpy as np
from functools import partial
from jax import lax
from jax.experimental import pallas as pl
from jax.experimental.pallas import tpu as pltpu
from jax.experimental.pallas import tpu_sc as plsc

T = 3
N = 4096
E = 131072
X_DIM = 128
H_DIM = 128
Z_DIM = 64
ALPHA = 0.2

_R = 512
_C = 1024
_ROWS = 256


def _gat_body(acts, m_ref, ht_ref, a1_ref, a2_ref, b1_ref, b2_ref, out_ref, rs_ref):
    cb = pl.program_id(1)
    ncb = pl.num_programs(1)
    G = ht_ref.shape[0]

    @pl.when(cb == 0)
    def _():
        out_ref[...] = jnp.zeros_like(out_ref)
        rs_ref[...] = jnp.zeros_like(rs_ref)

    m = m_ref[...]
    for k in range(G):
        w = jnp.minimum(a1_ref[k] * b1_ref[k], a2_ref[k] * b2_ref[k])
        p = m * w
        out_ref[k] += jnp.dot(p.astype(jnp.bfloat16), ht_ref[k],
                              preferred_element_type=jnp.float32)
        rs_ref[k] += jnp.sum(p, axis=1, keepdims=True)

    @pl.when(cb == ncb - 1)
    def _():
        for k in range(G):
            out_ref[k] = acts[k](out_ref[k] / rs_ref[k])


def _gat_pass(M, HT, FA, acts):
    G, n, D = HT.shape
    A1, A2, B1, B2 = FA
    grid = (N // _R, N // _C)
    row_spec = pl.BlockSpec((G, _R, 1), lambda rb, cb: (0, rb, 0))
    col_spec = pl.BlockSpec((G, 1, _C), lambda rb, cb: (0, 0, cb))
    return pl.pallas_call(
        partial(_gat_body, acts),
        grid=grid,
        in_specs=[
            pl.BlockSpec((_R, _C), lambda rb, cb: (rb, cb)),
            pl.BlockSpec((G, _C, D), lambda rb, cb: (0, cb, 0)),
            row_spec, row_spec, col_spec, col_spec,
        ],
        out_specs=pl.BlockSpec((G, _R, D), lambda rb, cb: (0, rb, 0)),
        out_shape=jax.ShapeDtypeStruct((G, N, D), jnp.float32),
        scratch_shapes=[pltpu.VMEM((G, _R, 1), jnp.float32)],
    )(M, HT, A1, A2, B1, B2)


def _factors(HT, A):
    G, n, D = HT.shape
    si = jnp.clip(jnp.einsum('gnd,gd->gn', HT, A[:, :D]), -60.0, 60.0)
    sj = jnp.clip(jnp.einsum('gnd,gd->gn', HT, A[:, D:]), -60.0, 60.0)
    a1 = jnp.exp(-si)[:, :, None]
    a2 = jnp.exp(-ALPHA * si)[:, :, None]
    b1 = jnp.exp(-sj)[:, None, :]
    b2 = jnp.exp(-ALPHA * sj)[:, None, :]
    return a1, a2, b1, b2


def _nll_body(adj_ref, h1_ref, h2_ref, out_ref):
    t = pl.program_id(0)
    b = pl.program_id(1)

    @pl.when(jnp.logical_and(t == 0, b == 0))
    def _():
        out_ref[...] = jnp.zeros_like(out_ref)

    clim = 16.118095651
    s = jnp.clip(h1_ref[0] + h2_ref[0], -clim, clim)
    adj = adj_ref[0]
    term = adj * s - jnp.maximum(s, 0.0) - jnp.log(1.0 + jnp.exp(-jnp.abs(s)))
    scale = 1.0 / (float(N) * float(N) * 8.0 * 128.0)
    out_ref[...] += jnp.sum(term) * scale


def _nll_all(adj, h1, h2):
    nb = N // _ROWS
    out = pl.pallas_call(
        _nll_body,
        grid=(T, nb),
        in_specs=[
            pl.BlockSpec((1, _ROWS, N), lambda t, b: (t, b, 0)),
            pl.BlockSpec((1, _ROWS, 1), lambda t, b: (t, b, 0)),
            pl.BlockSpec((1, 1, N), lambda t, b: (t, 0, 0)),
        ],
        out_specs=pl.BlockSpec((8, 128), lambda t, b: (0, 0)),
        out_shape=jax.ShapeDtypeStruct((8, 128), jnp.float32),
    )(adj, h1, h2)
    return -jnp.sum(out)


def _identity(v):
    return v


def _elu(v):
    return jnp.where(v > 0, v, jnp.exp(jnp.minimum(v, 0.0)) - 1.0)


def _softplus(v):
    return jnp.maximum(v, 0.0) + jnp.log(1.0 + jnp.exp(-jnp.abs(v)))


def _kld_gauss(m1, s1, m2, s2):
    eps = 1e-8
    kld = 2.0 * jnp.log(s2 + eps) - 2.0 * jnp.log(s1 + eps) + (s1 ** 2 + (m1 - m2) ** 2) / ((s2 + eps) ** 2) - 1.0
    return (0.5 / m1.shape[0]) * jnp.sum(kld)


_EPT = E // 16
_RWIN = 256
_NPASS = N // (2 * _RWIN)
_ACC = _RWIN * N
_TRASH = 16 * _EPT
_SHARE = _ACC // 16


def _mbuild_body(e0_hbm, e1_hbm, m_hbm, e0_v, e1_v, idx_v, val_v, zero_v, acc_sh):
    c = lax.axis_index("c")
    s = lax.axis_index("s")
    pltpu.sync_copy(e0_hbm.at[pl.ds(s * _EPT, _EPT)], e0_v)
    pltpu.sync_copy(e1_hbm.at[pl.ds(s * _EPT, _EPT)], e1_v)

    def zfill(i, _):
        zero_v[pl.ds(i * 16, 16)] = jnp.zeros((16,), jnp.float32)
        return 0
    lax.fori_loop(0, zero_v.shape[0] // 16, zfill, 0)

    def ofill(i, _):
        val_v[pl.ds(i * 16, 16)] = jnp.ones((16,), jnp.float32)
        return 0
    lax.fori_loop(0, val_v.shape[0] // 16, ofill, 0)

    for p in range(_NPASS):
        base = (p * 2 + c) * _RWIN

        def zcopy(i, _):
            pltpu.sync_copy(zero_v, acc_sh.at[pl.ds(s * _SHARE + i * 8192, 8192)])
            return 0
        lax.fori_loop(0, _SHARE // 8192, zcopy, 0)
        plsc.subcore_barrier()

        lanes = lax.iota(jnp.int32, 16)

        def estep(i, _):
            e0v = e0_v[pl.ds(i * 16, 16)]
            e1v = e1_v[pl.ds(i * 16, 16)]
            rel = e0v - base
            mask = (rel >= 0) & (rel < _RWIN)
            flat = rel * N + e1v
            junk = _ACC + s * _EPT + i * 16 + lanes
            idx_v[pl.ds(i * 16, 16)] = jnp.where(mask, flat, junk)
            return 0
        lax.fori_loop(0, _EPT // 16, estep, 0)

        rel_d = s * 16 + lax.iota(jnp.int32, 16)
        idx_v[pl.ds(_EPT, 16)] = rel_d * N + (base + rel_d)

        pltpu.sync_copy(val_v, acc_sh.at[idx_v], add=True)
        plsc.subcore_barrier()
        pltpu.sync_copy(acc_sh.at[pl.ds(s * _SHARE, _SHARE)],
                        m_hbm.at[pl.ds(base * N + s * _SHARE, _SHARE)])
        plsc.subcore_barrier()


_mbuild = partial(
    pl.kernel,
    out_type=jax.ShapeDtypeStruct((N * N,), jnp.float32),
    mesh=plsc.VectorSubcoreMesh(core_axis_name="c", subcore_axis_name="s"),
    scratch_types=[
        pltpu.VMEM((_EPT,), jnp.int32),
        pltpu.VMEM((_EPT,), jnp.int32),
        pltpu.VMEM((_EPT + 16,), jnp.int32),
        pltpu.VMEM((_EPT + 16,), jnp.float32),
        pltpu.VMEM((8192,), jnp.float32),
        pltpu.VMEM_SHARED((_ACC + _TRASH,), jnp.float32),
    ],
)(_mbuild_body)


def _build_m(edge):
    return _mbuild(edge[0], edge[1]).reshape(N, N)


def kernel(x, edge_idx_list, adj_orig_dense_list, phi_x_W, phi_x_b, phi_z_W, phi_z_b, enc_W, enc_a, enc_mu_W, enc_mu_a, enc_lv_W, enc_lv_a, prior_W, prior_b, prior_mu_W, prior_mu_b, prior_lv_W, prior_lv_b, lstm_x_W, lstm_x_a, lstm_h_W, lstm_h_a, dec_a):
    h = jnp.zeros((N, H_DIM), dtype=jnp.float32)
    c = jnp.zeros((N, H_DIM), dtype=jnp.float32)
    kld = jnp.float32(0.0)
    h1s, h2s = [], []
    for t in range(T):
        M = _build_m(edge_idx_list[t])
        phi_x_t = jax.nn.relu(x[t] @ phi_x_W + phi_x_b)

        h_enc = jnp.concatenate([phi_x_t, h], axis=1) @ enc_W
        HT1 = h_enc[None]
        enc_t = _gat_pass(M, HT1.astype(jnp.bfloat16), _factors(HT1, enc_a), [_elu])[0]

        HT2 = jnp.stack([enc_t @ enc_mu_W, enc_t @ enc_lv_W])
        A2 = jnp.stack([enc_mu_a[0], enc_lv_a[0]])
        o2 = _gat_pass(M, HT2.astype(jnp.bfloat16), _factors(HT2, A2), [_identity, _softplus])
        enc_mu_t, enc_std_t = o2[0], o2[1]

        prior_t = jax.nn.elu(h @ prior_W + prior_b)
        prior_mu_t = prior_t @ prior_mu_W + prior_mu_b
        prior_std_t = jax.nn.softplus(prior_t @ prior_lv_W + prior_lv_b)
        eps = jax.random.normal(jax.random.fold_in(jax.random.key(7), t), enc_mu_t.shape, dtype=jnp.float32)
        z_t = enc_mu_t + eps * enc_std_t
        phi_z_t = jax.nn.relu(z_t @ phi_z_W + phi_z_b)
        x_lstm = jnp.concatenate([phi_x_t, phi_z_t], axis=1)

        HT3 = jnp.concatenate([
            jnp.einsum('nk,gkd->gnd', x_lstm, lstm_x_W),
            jnp.einsum('nk,gkd->gnd', h, lstm_h_W),
        ])
        A3 = jnp.concatenate([lstm_x_a[:, 0, :], lstm_h_a[:, 0, :]])
        o3 = _gat_pass(M, HT3.astype(jnp.bfloat16), _factors(HT3, A3), [_identity] * 8)

        ig = jax.nn.sigmoid(o3[0] + o3[4])
        fg = jax.nn.sigmoid(o3[1] + o3[5])
        og = jax.nn.sigmoid(o3[2] + o3[6])
        ct = jnp.tanh(o3[3] + o3[7])
        c = fg * c + ig * ct
        h = og * jnp.tanh(c)
        kld = kld + _kld_gauss(enc_mu_t, enc_std_t, prior_mu_t, prior_std_t)
        h1s.append(z_t @ dec_a[:Z_DIM, :])
        h2s.append((z_t @ dec_a[Z_DIM:, :]).T)

    h1 = jnp.stack(h1s)
    h2 = jnp.stack(h2s)
    nll = _nll_all(adj_orig_dense_list, h1, h2)
    return jnp.stack([kld, nll])


# speedup vs baseline: 3.1678x; 1.0745x over previous
import jax
import jax.numpy as jnp
import numpy as np
from functools import partial
from jax import lax
from jax.experimental import pallas as pl
from jax.experimental.pallas import tpu as pltpu
from jax.experimental.pallas import tpu_sc as plsc

T = 3
N = 4096
E = 131072
X_DIM = 128
H_DIM = 128
Z_DIM = 64
ALPHA = 0.2

_R = 512
_C = 1024
_ROWS = 256


def _gat_body(acts, m_ref, ht_ref, a3_ref, b1_ref, b2_ref, out_ref, rs_ref):
    cb = pl.program_id(1)
    ncb = pl.num_programs(1)
    G = ht_ref.shape[0]

    @pl.when(cb == 0)
    def _():
        out_ref[...] = jnp.zeros_like(out_ref)
        rs_ref[...] = jnp.zeros_like(rs_ref)

    m = m_ref[...]
    for k in range(G):
        w = jnp.minimum(a3_ref[k] * b1_ref[k], b2_ref[k])
        p = m * w
        out_ref[k] += jnp.dot(p.astype(jnp.bfloat16), ht_ref[k],
                              preferred_element_type=jnp.float32)
        rs_ref[k] += jnp.sum(p, axis=1, keepdims=True)

    @pl.when(cb == ncb - 1)
    def _():
        for k in range(G):
            out_ref[k] = acts[k](out_ref[k] / rs_ref[k])


def _gat_pass(M, HT, FA, acts):
    G, n, D = HT.shape
    A3, B1, B2 = FA
    grid = (N // _R, N // _C)
    row_spec = pl.BlockSpec((G, _R, 1), lambda rb, cb: (0, rb, 0))
    col_spec = pl.BlockSpec((G, 1, _C), lambda rb, cb: (0, 0, cb))
    return pl.pallas_call(
        partial(_gat_body, acts),
        grid=grid,
        in_specs=[
            pl.BlockSpec((_R, _C), lambda rb, cb: (rb, cb)),
            pl.BlockSpec((G, _C, D), lambda rb, cb: (0, cb, 0)),
            row_spec, col_spec, col_spec,
        ],
        out_specs=pl.BlockSpec((G, _R, D), lambda rb, cb: (0, rb, 0)),
        out_shape=jax.ShapeDtypeStruct((G, N, D), jnp.float32),
        scratch_shapes=[pltpu.VMEM((G, _R, 1), jnp.float32)],
    )(M, HT, A3, B1, B2)


def _factors(HT, A):
    G, n, D = HT.shape
    si = jnp.clip(jnp.einsum('gnd,gd->gn', HT, A[:, :D]), -60.0, 60.0)
    sj = jnp.clip(jnp.einsum('gnd,gd->gn', HT, A[:, D:]), -60.0, 60.0)
    a3 = jnp.exp(-(1.0 - ALPHA) * si)[:, :, None]
    b1 = jnp.exp(-sj)[:, None, :]
    b2 = jnp.exp(-ALPHA * sj)[:, None, :]
    return a3, b1, b2


def _nll_body(adj_ref, h1_ref, h2_ref, out_ref):
    t = pl.program_id(0)
    b = pl.program_id(1)

    @pl.when(jnp.logical_and(t == 0, b == 0))
    def _():
        out_ref[...] = jnp.zeros_like(out_ref)

    clim = 16.118095651
    s = jnp.clip(h1_ref[0] + h2_ref[0], -clim, clim)
    adj = adj_ref[0]
    term = adj * s - jnp.maximum(s, 0.0) - jnp.log(1.0 + jnp.exp(-jnp.abs(s)))
    scale = 1.0 / (float(N) * float(N) * 8.0 * 128.0)
    out_ref[...] += jnp.sum(term) * scale


def _nll_all(adj, h1, h2):
    nb = N // _ROWS
    out = pl.pallas_call(
        _nll_body,
        grid=(T, nb),
        in_specs=[
            pl.BlockSpec((1, _ROWS, N), lambda t, b: (t, b, 0)),
            pl.BlockSpec((1, _ROWS, 1), lambda t, b: (t, b, 0)),
            pl.BlockSpec((1, 1, N), lambda t, b: (t, 0, 0)),
        ],
        out_specs=pl.BlockSpec((8, 128), lambda t, b: (0, 0)),
        out_shape=jax.ShapeDtypeStruct((8, 128), jnp.float32),
    )(adj, h1, h2)
    return -jnp.sum(out)


def _identity(v):
    return v


def _elu(v):
    return jnp.where(v > 0, v, jnp.exp(jnp.minimum(v, 0.0)) - 1.0)


def _softplus(v):
    return jnp.maximum(v, 0.0) + jnp.log(1.0 + jnp.exp(-jnp.abs(v)))


def _kld_gauss(m1, s1, m2, s2):
    eps = 1e-8
    kld = 2.0 * jnp.log(s2 + eps) - 2.0 * jnp.log(s1 + eps) + (s1 ** 2 + (m1 - m2) ** 2) / ((s2 + eps) ** 2) - 1.0
    return (0.5 / m1.shape[0]) * jnp.sum(kld)


_EPT = E // 16
_RWIN = 256
_NPASS = N // (2 * _RWIN)
_ACC = _RWIN * N
_TRASH = 16 * _EPT
_SHARE = _ACC // 16


def _mbuild_body(e0_hbm, e1_hbm, m_hbm, e0_v, e1_v, idx_v, val_v, zero_v, acc_sh):
    c = lax.axis_index("c")
    s = lax.axis_index("s")
    pltpu.sync_copy(e0_hbm.at[pl.ds(s * _EPT, _EPT)], e0_v)
    pltpu.sync_copy(e1_hbm.at[pl.ds(s * _EPT, _EPT)], e1_v)

    def zfill(i, _):
        zero_v[pl.ds(i * 16, 16)] = jnp.zeros((16,), jnp.float32)
        return 0
    lax.fori_loop(0, zero_v.shape[0] // 16, zfill, 0)

    def ofill(i, _):
        val_v[pl.ds(i * 16, 16)] = jnp.ones((16,), jnp.float32)
        return 0
    lax.fori_loop(0, val_v.shape[0] // 16, ofill, 0)

    for p in range(_NPASS):
        base = (p * 2 + c) * _RWIN

        def zcopy(i, _):
            pltpu.sync_copy(zero_v, acc_sh.at[pl.ds(s * _SHARE + i * 8192, 8192)])
            return 0
        lax.fori_loop(0, _SHARE // 8192, zcopy, 0)
        plsc.subcore_barrier()

        lanes = lax.iota(jnp.int32, 16)

        def estep(i, _):
            e0v = e0_v[pl.ds(i * 16, 16)]
            e1v = e1_v[pl.ds(i * 16, 16)]
            rel = e0v - base
            mask = (rel >= 0) & (rel < _RWIN)
            flat = rel * N + e1v
            junk = _ACC + s * _EPT + i * 16 + lanes
            idx_v[pl.ds(i * 16, 16)] = jnp.where(mask, flat, junk)
            return 0
        lax.fori_loop(0, _EPT // 16, estep, 0)

        rel_d = s * 16 + lax.iota(jnp.int32, 16)
        idx_v[pl.ds(_EPT, 16)] = rel_d * N + (base + rel_d)

        pltpu.sync_copy(val_v, acc_sh.at[idx_v], add=True)
        plsc.subcore_barrier()
        pltpu.sync_copy(acc_sh.at[pl.ds(s * _SHARE, _SHARE)],
                        m_hbm.at[pl.ds(base * N + s * _SHARE, _SHARE)])
        plsc.subcore_barrier()


_mbuild = partial(
    pl.kernel,
    out_type=jax.ShapeDtypeStruct((N * N,), jnp.float32),
    mesh=plsc.VectorSubcoreMesh(core_axis_name="c", subcore_axis_name="s"),
    scratch_types=[
        pltpu.VMEM((_EPT,), jnp.int32),
        pltpu.VMEM((_EPT,), jnp.int32),
        pltpu.VMEM((_EPT + 16,), jnp.int32),
        pltpu.VMEM((_EPT + 16,), jnp.float32),
        pltpu.VMEM((8192,), jnp.float32),
        pltpu.VMEM_SHARED((_ACC + _TRASH,), jnp.float32),
    ],
)(_mbuild_body)


def _build_m(edge):
    return _mbuild(edge[0], edge[1]).reshape(N, N)


def kernel(x, edge_idx_list, adj_orig_dense_list, phi_x_W, phi_x_b, phi_z_W, phi_z_b, enc_W, enc_a, enc_mu_W, enc_mu_a, enc_lv_W, enc_lv_a, prior_W, prior_b, prior_mu_W, prior_mu_b, prior_lv_W, prior_lv_b, lstm_x_W, lstm_x_a, lstm_h_W, lstm_h_a, dec_a):
    h = jnp.zeros((N, H_DIM), dtype=jnp.float32)
    c = jnp.zeros((N, H_DIM), dtype=jnp.float32)
    kld = jnp.float32(0.0)
    h1s, h2s = [], []
    for t in range(T):
        M = _build_m(edge_idx_list[t])
        phi_x_t = jax.nn.relu(x[t] @ phi_x_W + phi_x_b)

        h_enc = jnp.concatenate([phi_x_t, h], axis=1) @ enc_W
        HT1 = h_enc[None]
        enc_t = _gat_pass(M, HT1.astype(jnp.bfloat16), _factors(HT1, enc_a), [_elu])[0]

        HT2 = jnp.stack([enc_t @ enc_mu_W, enc_t @ enc_lv_W])
        A2 = jnp.stack([enc_mu_a[0], enc_lv_a[0]])
        o2 = _gat_pass(M, HT2.astype(jnp.bfloat16), _factors(HT2, A2), [_identity, _softplus])
        enc_mu_t, enc_std_t = o2[0], o2[1]

        prior_t = jax.nn.elu(h @ prior_W + prior_b)
        prior_mu_t = prior_t @ prior_mu_W + prior_mu_b
        prior_std_t = jax.nn.softplus(prior_t @ prior_lv_W + prior_lv_b)
        eps = jax.random.normal(jax.random.fold_in(jax.random.key(7), t), enc_mu_t.shape, dtype=jnp.float32)
        z_t = enc_mu_t + eps * enc_std_t
        phi_z_t = jax.nn.relu(z_t @ phi_z_W + phi_z_b)
        x_lstm = jnp.concatenate([phi_x_t, phi_z_t], axis=1)

        HT3 = jnp.concatenate([
            jnp.einsum('nk,gkd->gnd', x_lstm, lstm_x_W),
            jnp.einsum('nk,gkd->gnd', h, lstm_h_W),
        ])
        A3 = jnp.concatenate([lstm_x_a[:, 0, :], lstm_h_a[:, 0, :]])
        o3 = _gat_pass(M, HT3.astype(jnp.bfloat16), _factors(HT3, A3), [_identity] * 8)

        ig = jax.nn.sigmoid(o3[0] + o3[4])
        fg = jax.nn.sigmoid(o3[1] + o3[5])
        og = jax.nn.sigmoid(o3[2] + o3[6])
        ct = jnp.tanh(o3[3] + o3[7])
        c = fg * c + ig * ct
        h = og * jnp.tanh(c)
        kld = kld + _kld_gauss(enc_mu_t, enc_std_t, prior_mu_t, prior_std_t)
        h1s.append(z_t @ dec_a[:Z_DIM, :])
        h2s.append((z_t @ dec_a[Z_DIM:, :]).T)

    h1 = jnp.stack(h1s)
    h2 = jnp.stack(h2s)
    nll = _nll_all(adj_orig_dense_list, h1, h2)
    return jnp.stack([kld, nll])


# speedup vs baseline: 3.1680x; 1.0001x over previous
import jax
import jax.numpy as jnp
import numpy as np
from functools import partial
from jax import lax
from jax.experimental import pallas as pl
from jax.experimental.pallas import tpu as pltpu
from jax.experimental.pallas import tpu_sc as plsc

T = 3
N = 4096
E = 131072
X_DIM = 128
H_DIM = 128
Z_DIM = 64
ALPHA = 0.2

_R = 512
_C = 1024
_ROWS = 256


def _gat_body(acts, m_ref, ht_ref, a3_ref, b1_ref, b2_ref, out_ref, rs_ref):
    cb = pl.program_id(1)
    ncb = pl.num_programs(1)
    G = ht_ref.shape[0]

    @pl.when(cb == 0)
    def _():
        out_ref[...] = jnp.zeros_like(out_ref)
        rs_ref[...] = jnp.zeros_like(rs_ref)

    m = m_ref[...]
    for k in range(G):
        w = jnp.minimum(a3_ref[k] * b1_ref[k], b2_ref[k])
        p = m * w
        out_ref[k] += jnp.dot(p.astype(jnp.bfloat16), ht_ref[k],
                              preferred_element_type=jnp.float32)
        rs_ref[k] += jnp.sum(p, axis=1, keepdims=True)

    @pl.when(cb == ncb - 1)
    def _():
        for k in range(G):
            out_ref[k] = acts[k](out_ref[k] / rs_ref[k])


def _gat_pass(M, HT, FA, acts):
    G, n, D = HT.shape
    A3, B1, B2 = FA
    grid = (N // _R, N // _C)
    row_spec = pl.BlockSpec((G, _R, 1), lambda rb, cb: (0, rb, 0))
    col_spec = pl.BlockSpec((G, 1, _C), lambda rb, cb: (0, 0, cb))
    return pl.pallas_call(
        partial(_gat_body, acts),
        grid=grid,
        in_specs=[
            pl.BlockSpec((_R, _C), lambda rb, cb: (rb, cb)),
            pl.BlockSpec((G, _C, D), lambda rb, cb: (0, cb, 0)),
            row_spec, col_spec, col_spec,
        ],
        out_specs=pl.BlockSpec((G, _R, D), lambda rb, cb: (0, rb, 0)),
        out_shape=jax.ShapeDtypeStruct((G, N, D), jnp.float32),
        scratch_shapes=[pltpu.VMEM((G, _R, 1), jnp.float32)],
    )(M, HT, A3, B1, B2)


def _factors(HT, A):
    G, n, D = HT.shape
    si = jnp.clip(jnp.einsum('gnd,gd->gn', HT, A[:, :D]), -60.0, 60.0)
    sj = jnp.clip(jnp.einsum('gnd,gd->gn', HT, A[:, D:]), -60.0, 60.0)
    a3 = jnp.exp(-(1.0 - ALPHA) * si)[:, :, None]
    b1 = jnp.exp(-sj)[:, None, :]
    b2 = jnp.exp(-ALPHA * sj)[:, None, :]
    return a3, b1, b2


def _nll_body(adj_ref, h1_ref, h2_ref, out_ref):
    t = pl.program_id(0)
    b = pl.program_id(1)

    @pl.when(jnp.logical_and(t == 0, b == 0))
    def _():
        out_ref[...] = jnp.zeros_like(out_ref)

    clim = 16.118095651
    s = jnp.clip(h1_ref[0] + h2_ref[0], -clim, clim)
    adj = adj_ref[0]
    term = adj * s - jnp.maximum(s, 0.0) - jnp.log(1.0 + jnp.exp(-jnp.abs(s)))
    scale = 1.0 / (float(N) * float(N) * 8.0 * 128.0)
    out_ref[...] += jnp.sum(term) * scale


def _nll_all(adj, h1, h2):
    nb = N // _ROWS
    out = pl.pallas_call(
        _nll_body,
        grid=(T, nb),
        in_specs=[
            pl.BlockSpec((1, _ROWS, N), lambda t, b: (t, b, 0)),
            pl.BlockSpec((1, _ROWS, 1), lambda t, b: (t, b, 0)),
            pl.BlockSpec((1, 1, N), lambda t, b: (t, 0, 0)),
        ],
        out_specs=pl.BlockSpec((8, 128), lambda t, b: (0, 0)),
        out_shape=jax.ShapeDtypeStruct((8, 128), jnp.float32),
    )(adj, h1, h2)
    return -jnp.sum(out)


def _identity(v):
    return v


def _elu(v):
    return jnp.where(v > 0, v, jnp.exp(jnp.minimum(v, 0.0)) - 1.0)


def _softplus(v):
    return jnp.maximum(v, 0.0) + jnp.log(1.0 + jnp.exp(-jnp.abs(v)))


def _kld_gauss(m1, s1, m2, s2):
    eps = 1e-8
    kld = 2.0 * jnp.log(s2 + eps) - 2.0 * jnp.log(s1 + eps) + (s1 ** 2 + (m1 - m2) ** 2) / ((s2 + eps) ** 2) - 1.0
    return (0.5 / m1.shape[0]) * jnp.sum(kld)


_EPT = E // 16
_RWIN = 256
_NPASS = N // (2 * _RWIN)
_ACC = _RWIN * N
_TRASH = 16 * _EPT
_SHARE = _ACC // 16


def _mbuild_body(e0_hbm, e1_hbm, m_hbm, e0_v, e1_v, idx_v, val_v, zero_v, acc_sh):
    c = lax.axis_index("c")
    s = lax.axis_index("s")
    pltpu.sync_copy(e0_hbm.at[pl.ds(s * _EPT, _EPT)], e0_v)
    pltpu.sync_copy(e1_hbm.at[pl.ds(s * _EPT, _EPT)], e1_v)

    def zfill(i, _):
        zero_v[pl.ds(i * 16, 16)] = jnp.zeros((16,), jnp.float32)
        return 0
    lax.fori_loop(0, zero_v.shape[0] // 16, zfill, 0)

    def ofill(i, _):
        val_v[pl.ds(i * 16, 16)] = jnp.ones((16,), jnp.float32)
        return 0
    lax.fori_loop(0, val_v.shape[0] // 16, ofill, 0)

    for p in range(_NPASS):
        base = (p * 2 + c) * _RWIN

        def zcopy(i, _):
            pltpu.sync_copy(zero_v, acc_sh.at[pl.ds(s * _SHARE + i * 8192, 8192)])
            return 0
        lax.fori_loop(0, _SHARE // 8192, zcopy, 0)
        plsc.subcore_barrier()

        lanes = lax.iota(jnp.int32, 16)

        def estep(i, _):
            e0v = e0_v[pl.ds(i * 16, 16)]
            e1v = e1_v[pl.ds(i * 16, 16)]
            rel = e0v - base
            mask = (rel >= 0) & (rel < _RWIN)
            flat = rel * N + e1v
            junk = _ACC + s * _EPT + i * 16 + lanes
            idx_v[pl.ds(i * 16, 16)] = jnp.where(mask, flat, junk)
            return 0
        lax.fori_loop(0, _EPT // 16, estep, 0)

        rel_d = s * 16 + lax.iota(jnp.int32, 16)
        idx_v[pl.ds(_EPT, 16)] = rel_d * N + (base + rel_d)

        pltpu.sync_copy(val_v, acc_sh.at[idx_v], add=True)
        plsc.subcore_barrier()
        pltpu.sync_copy(acc_sh.at[pl.ds(s * _SHARE, _SHARE)],
                        m_hbm.at[pl.ds(base * N + s * _SHARE, _SHARE)])
        plsc.subcore_barrier()


_mbuild = partial(
    pl.kernel,
    out_type=jax.ShapeDtypeStruct((N * N,), jnp.float32),
    mesh=plsc.VectorSubcoreMesh(core_axis_name="c", subcore_axis_name="s"),
    scratch_types=[
        pltpu.VMEM((_EPT,), jnp.int32),
        pltpu.VMEM((_EPT,), jnp.int32),
        pltpu.VMEM((_EPT + 16,), jnp.int32),
        pltpu.VMEM((_EPT + 16,), jnp.float32),
        pltpu.VMEM((8192,), jnp.float32),
        pltpu.VMEM_SHARED((_ACC + _TRASH,), jnp.float32),
    ],
)(_mbuild_body)


def _build_m(edge):
    return _mbuild(edge[0], edge[1]).reshape(N, N)


def kernel(x, edge_idx_list, adj_orig_dense_list, phi_x_W, phi_x_b, phi_z_W, phi_z_b, enc_W, enc_a, enc_mu_W, enc_mu_a, enc_lv_W, enc_lv_a, prior_W, prior_b, prior_mu_W, prior_mu_b, prior_lv_W, prior_lv_b, lstm_x_W, lstm_x_a, lstm_h_W, lstm_h_a, dec_a):
    h = jnp.zeros((N, H_DIM), dtype=jnp.float32)
    c = jnp.zeros((N, H_DIM), dtype=jnp.float32)
    kld = jnp.float32(0.0)
    h1s, h2s = [], []
    Ms = [_build_m(edge_idx_list[t]) for t in range(T)]
    for t in range(T):
        M = Ms[t]
        phi_x_t = jax.nn.relu(x[t] @ phi_x_W + phi_x_b)

        h_enc = jnp.concatenate([phi_x_t, h], axis=1) @ enc_W
        HT1 = h_enc[None]
        enc_t = _gat_pass(M, HT1.astype(jnp.bfloat16), _factors(HT1, enc_a), [_elu])[0]

        HT2 = jnp.stack([enc_t @ enc_mu_W, enc_t @ enc_lv_W])
        A2 = jnp.stack([enc_mu_a[0], enc_lv_a[0]])
        o2 = _gat_pass(M, HT2.astype(jnp.bfloat16), _factors(HT2, A2), [_identity, _softplus])
        enc_mu_t, enc_std_t = o2[0], o2[1]

        prior_t = jax.nn.elu(h @ prior_W + prior_b)
        prior_mu_t = prior_t @ prior_mu_W + prior_mu_b
        prior_std_t = jax.nn.softplus(prior_t @ prior_lv_W + prior_lv_b)
        eps = jax.random.normal(jax.random.fold_in(jax.random.key(7), t), enc_mu_t.shape, dtype=jnp.float32)
        z_t = enc_mu_t + eps * enc_std_t
        phi_z_t = jax.nn.relu(z_t @ phi_z_W + phi_z_b)
        x_lstm = jnp.concatenate([phi_x_t, phi_z_t], axis=1)

        HT3 = jnp.concatenate([
            jnp.einsum('nk,gkd->gnd', x_lstm, lstm_x_W),
            jnp.einsum('nk,gkd->gnd', h, lstm_h_W),
        ])
        A3 = jnp.concatenate([lstm_x_a[:, 0, :], lstm_h_a[:, 0, :]])
        o3 = _gat_pass(M, HT3.astype(jnp.bfloat16), _factors(HT3, A3), [_identity] * 8)

        ig = jax.nn.sigmoid(o3[0] + o3[4])
        fg = jax.nn.sigmoid(o3[1] + o3[5])
        og = jax.nn.sigmoid(o3[2] + o3[6])
        ct = jnp.tanh(o3[3] + o3[7])
        c = fg * c + ig * ct
        h = og * jnp.tanh(c)
        kld = kld + _kld_gauss(enc_mu_t, enc_std_t, prior_mu_t, prior_std_t)
        h1s.append(z_t @ dec_a[:Z_DIM, :])
        h2s.append((z_t @ dec_a[Z_DIM:, :]).T)

    h1 = jnp.stack(h1s)
    h2 = jnp.stack(h2s)
    nll = _nll_all(adj_orig_dense_list, h1, h2)
    return jnp.stack([kld, nll])


# speedup vs baseline: 3.2452x; 1.0244x over previous
import jax
import jax.numpy as jnp
import numpy as np
from functools import partial
from jax import lax
from jax.experimental import pallas as pl
from jax.experimental.pallas import tpu as pltpu
from jax.experimental.pallas import tpu_sc as plsc

T = 3
N = 4096
E = 131072
X_DIM = 128
H_DIM = 128
Z_DIM = 64
ALPHA = 0.2

_R = 512
_C = 2048
_ROWS = 256


def _gat_body(acts, m_ref, ht_ref, a3_ref, b1_ref, b2_ref, out_ref, rs_ref):
    cb = pl.program_id(1)
    ncb = pl.num_programs(1)
    G = ht_ref.shape[0]

    @pl.when(cb == 0)
    def _():
        out_ref[...] = jnp.zeros_like(out_ref)
        rs_ref[...] = jnp.zeros_like(rs_ref)

    m = m_ref[...]
    for k in range(G):
        w = jnp.minimum(a3_ref[k] * b1_ref[k], b2_ref[k])
        p = m * w
        out_ref[k] += jnp.dot(p.astype(jnp.bfloat16), ht_ref[k],
                              preferred_element_type=jnp.float32)
        rs_ref[k] += jnp.sum(p, axis=1, keepdims=True)

    @pl.when(cb == ncb - 1)
    def _():
        for k in range(G):
            out_ref[k] = acts[k](out_ref[k] / rs_ref[k])


def _gat_pass(M, HT, FA, acts):
    G, n, D = HT.shape
    A3, B1, B2 = FA
    grid = (N // _R, N // _C)
    row_spec = pl.BlockSpec((G, _R, 1), lambda rb, cb: (0, rb, 0))
    col_spec = pl.BlockSpec((G, 1, _C), lambda rb, cb: (0, 0, cb))
    return pl.pallas_call(
        partial(_gat_body, acts),
        grid=grid,
        in_specs=[
            pl.BlockSpec((_R, _C), lambda rb, cb: (rb, cb)),
            pl.BlockSpec((G, _C, D), lambda rb, cb: (0, cb, 0)),
            row_spec, col_spec, col_spec,
        ],
        out_specs=pl.BlockSpec((G, _R, D), lambda rb, cb: (0, rb, 0)),
        out_shape=jax.ShapeDtypeStruct((G, N, D), jnp.float32),
        scratch_shapes=[pltpu.VMEM((G, _R, 1), jnp.float32)],
    )(M, HT, A3, B1, B2)


def _factors(HT, A):
    G, n, D = HT.shape
    si = jnp.clip(jnp.einsum('gnd,gd->gn', HT, A[:, :D]), -60.0, 60.0)
    sj = jnp.clip(jnp.einsum('gnd,gd->gn', HT, A[:, D:]), -60.0, 60.0)
    a3 = jnp.exp(-(1.0 - ALPHA) * si)[:, :, None]
    b1 = jnp.exp(-sj)[:, None, :]
    b2 = jnp.exp(-ALPHA * sj)[:, None, :]
    return a3, b1, b2


def _nll_body(adj_ref, h1_ref, h2_ref, out_ref):
    t = pl.program_id(0)
    b = pl.program_id(1)

    @pl.when(jnp.logical_and(t == 0, b == 0))
    def _():
        out_ref[...] = jnp.zeros_like(out_ref)

    clim = 16.118095651
    s = jnp.clip(h1_ref[0] + h2_ref[0], -clim, clim)
    adj = adj_ref[0]
    term = adj * s - jnp.maximum(s, 0.0) - jnp.log(1.0 + jnp.exp(-jnp.abs(s)))
    scale = 1.0 / (float(N) * float(N) * 8.0 * 128.0)
    out_ref[...] += jnp.sum(term) * scale


def _nll_all(adj, h1, h2):
    nb = N // _ROWS
    out = pl.pallas_call(
        _nll_body,
        grid=(T, nb),
        in_specs=[
            pl.BlockSpec((1, _ROWS, N), lambda t, b: (t, b, 0)),
            pl.BlockSpec((1, _ROWS, 1), lambda t, b: (t, b, 0)),
            pl.BlockSpec((1, 1, N), lambda t, b: (t, 0, 0)),
        ],
        out_specs=pl.BlockSpec((8, 128), lambda t, b: (0, 0)),
        out_shape=jax.ShapeDtypeStruct((8, 128), jnp.float32),
    )(adj, h1, h2)
    return -jnp.sum(out)


def _identity(v):
    return v


def _elu(v):
    return jnp.where(v > 0, v, jnp.exp(jnp.minimum(v, 0.0)) - 1.0)


def _softplus(v):
    return jnp.maximum(v, 0.0) + jnp.log(1.0 + jnp.exp(-jnp.abs(v)))


def _kld_gauss(m1, s1, m2, s2):
    eps = 1e-8
    kld = 2.0 * jnp.log(s2 + eps) - 2.0 * jnp.log(s1 + eps) + (s1 ** 2 + (m1 - m2) ** 2) / ((s2 + eps) ** 2) - 1.0
    return (0.5 / m1.shape[0]) * jnp.sum(kld)


_EPT = E // 16
_RWIN = 256
_NPASS = N // (2 * _RWIN)
_ACC = _RWIN * N
_TRASH = 16 * _EPT
_SHARE = _ACC // 16


def _mbuild_body(e0_hbm, e1_hbm, m_hbm, e0_v, e1_v, idx_v, val_v, zero_v, acc_sh):
    c = lax.axis_index("c")
    s = lax.axis_index("s")
    pltpu.sync_copy(e0_hbm.at[pl.ds(s * _EPT, _EPT)], e0_v)
    pltpu.sync_copy(e1_hbm.at[pl.ds(s * _EPT, _EPT)], e1_v)

    def zfill(i, _):
        zero_v[pl.ds(i * 16, 16)] = jnp.zeros((16,), jnp.float32)
        return 0
    lax.fori_loop(0, zero_v.shape[0] // 16, zfill, 0)

    def ofill(i, _):
        val_v[pl.ds(i * 16, 16)] = jnp.ones((16,), jnp.float32)
        return 0
    lax.fori_loop(0, val_v.shape[0] // 16, ofill, 0)

    for p in range(_NPASS):
        base = (p * 2 + c) * _RWIN

        def zcopy(i, _):
            pltpu.sync_copy(zero_v, acc_sh.at[pl.ds(s * _SHARE + i * 8192, 8192)])
            return 0
        lax.fori_loop(0, _SHARE // 8192, zcopy, 0)
        plsc.subcore_barrier()

        lanes = lax.iota(jnp.int32, 16)

        def estep(i, _):
            e0v = e0_v[pl.ds(i * 16, 16)]
            e1v = e1_v[pl.ds(i * 16, 16)]
            rel = e0v - base
            mask = (rel >= 0) & (rel < _RWIN)
            flat = rel * N + e1v
            junk = _ACC + s * _EPT + i * 16 + lanes
            idx_v[pl.ds(i * 16, 16)] = jnp.where(mask, flat, junk)
            return 0
        lax.fori_loop(0, _EPT // 16, estep, 0)

        rel_d = s * 16 + lax.iota(jnp.int32, 16)
        idx_v[pl.ds(_EPT, 16)] = rel_d * N + (base + rel_d)

        pltpu.sync_copy(val_v, acc_sh.at[idx_v], add=True)
        plsc.subcore_barrier()
        pltpu.sync_copy(acc_sh.at[pl.ds(s * _SHARE, _SHARE)],
                        m_hbm.at[pl.ds(base * N + s * _SHARE, _SHARE)])
        plsc.subcore_barrier()


_mbuild = partial(
    pl.kernel,
    out_type=jax.ShapeDtypeStruct((N * N,), jnp.float32),
    mesh=plsc.VectorSubcoreMesh(core_axis_name="c", subcore_axis_name="s"),
    scratch_types=[
        pltpu.VMEM((_EPT,), jnp.int32),
        pltpu.VMEM((_EPT,), jnp.int32),
        pltpu.VMEM((_EPT + 16,), jnp.int32),
        pltpu.VMEM((_EPT + 16,), jnp.float32),
        pltpu.VMEM((8192,), jnp.float32),
        pltpu.VMEM_SHARED((_ACC + _TRASH,), jnp.float32),
    ],
)(_mbuild_body)


def _build_m(edge):
    return _mbuild(edge[0], edge[1]).reshape(N, N)


def kernel(x, edge_idx_list, adj_orig_dense_list, phi_x_W, phi_x_b, phi_z_W, phi_z_b, enc_W, enc_a, enc_mu_W, enc_mu_a, enc_lv_W, enc_lv_a, prior_W, prior_b, prior_mu_W, prior_mu_b, prior_lv_W, prior_lv_b, lstm_x_W, lstm_x_a, lstm_h_W, lstm_h_a, dec_a):
    h = jnp.zeros((N, H_DIM), dtype=jnp.float32)
    c = jnp.zeros((N, H_DIM), dtype=jnp.float32)
    kld = jnp.float32(0.0)
    h1s, h2s = [], []
    Ms = [_build_m(edge_idx_list[t]) for t in range(T)]
    for t in range(T):
        M = Ms[t]
        phi_x_t = jax.nn.relu(x[t] @ phi_x_W + phi_x_b)

        h_enc = jnp.concatenate([phi_x_t, h], axis=1) @ enc_W
        HT1 = h_enc[None]
        enc_t = _gat_pass(M, HT1.astype(jnp.bfloat16), _factors(HT1, enc_a), [_elu])[0]

        HT2 = jnp.stack([enc_t @ enc_mu_W, enc_t @ enc_lv_W])
        A2 = jnp.stack([enc_mu_a[0], enc_lv_a[0]])
        o2 = _gat_pass(M, HT2.astype(jnp.bfloat16), _factors(HT2, A2), [_identity, _softplus])
        enc_mu_t, enc_std_t = o2[0], o2[1]

        prior_t = jax.nn.elu(h @ prior_W + prior_b)
        prior_mu_t = prior_t @ prior_mu_W + prior_mu_b
        prior_std_t = jax.nn.softplus(prior_t @ prior_lv_W + prior_lv_b)
        eps = jax.random.normal(jax.random.fold_in(jax.random.key(7), t), enc_mu_t.shape, dtype=jnp.float32)
        z_t = enc_mu_t + eps * enc_std_t
        phi_z_t = jax.nn.relu(z_t @ phi_z_W + phi_z_b)
        x_lstm = jnp.concatenate([phi_x_t, phi_z_t], axis=1)

        HT3 = jnp.concatenate([
            jnp.einsum('nk,gkd->gnd', x_lstm, lstm_x_W),
            jnp.einsum('nk,gkd->gnd', h, lstm_h_W),
        ])
        A3 = jnp.concatenate([lstm_x_a[:, 0, :], lstm_h_a[:, 0, :]])
        o3 = _gat_pass(M, HT3.astype(jnp.bfloat16), _factors(HT3, A3), [_identity] * 8)

        ig = jax.nn.sigmoid(o3[0] + o3[4])
        fg = jax.nn.sigmoid(o3[1] + o3[5])
        og = jax.nn.sigmoid(o3[2] + o3[6])
        ct = jnp.tanh(o3[3] + o3[7])
        c = fg * c + ig * ct
        h = og * jnp.tanh(c)
        kld = kld + _kld_gauss(enc_mu_t, enc_std_t, prior_mu_t, prior_std_t)
        h1s.append(z_t @ dec_a[:Z_DIM, :])
        h2s.append((z_t @ dec_a[Z_DIM:, :]).T)

    h1 = jnp.stack(h1s)
    h2 = jnp.stack(h2s)
    nll = _nll_all(adj_orig_dense_list, h1, h2)
    return jnp.stack([kld, nll])
